# Initial kernel scaffold; baseline (speedup 1.0000x reference)
import jax
import jax.numpy as jnp
from jax.experimental import pallas as pl


def kernel(mask, hor_values, ver_values, x, hor_indices, ver_indices):
    raise NotImplementedError("write your pallas kernel here")


# speedup vs baseline: 6.3123x; 6.3123x over previous
import functools

import jax
import jax.numpy as jnp
from jax import lax
from jax.experimental import pallas as pl
from jax.experimental.pallas import tpu as pltpu
from jax.experimental.pallas import tpu_sc as plsc

N = 10000
E = 320000
D = 128

NS = 16
L = 16

C = 80
EPT = E // NS
CPS = 50
NSUP = EPT // (CPS * C)
RA = 624
RLAST = N - (NS - 1) * RA

_mesh = plsc.VectorSubcoreMesh(core_axis_name="c", subcore_axis_name="s")

_GATHER_DNUMS = lax.GatherDimensionNumbers(
    offset_dims=(), collapsed_slice_dims=(0,), start_index_map=(0,))


def _lane_broadcast(vec, i):
    idx = jnp.full((L, 1), i, jnp.int32)
    return lax.gather(vec, idx, _GATHER_DNUMS, slice_sizes=(1,),
                      mode=lax.GatherScatterMode.PROMISE_IN_BOUNDS)


@functools.partial(
    pl.kernel,
    out_type=[
        jax.ShapeDtypeStruct((N, D), jnp.float32),
        jax.ShapeDtypeStruct((N, D), jnp.float32),
    ],
    mesh=_mesh,
    scratch_types=[
        pltpu.VMEM((CPS, C), jnp.float32),
        pltpu.VMEM((CPS, C), jnp.float32),
        pltpu.VMEM((CPS, C), jnp.int32),
        pltpu.VMEM((CPS, C), jnp.int32),
        pltpu.VMEM((C, D), jnp.float32),
        pltpu.VMEM_SHARED((N, D), jnp.float32),
        pltpu.SemaphoreType.DMA,
    ],
)
def _sc_propagate(mask5, vals5, cols5, rows5, x_hbm, z_hbm, hor_out, ver_out,
                  mask_s, vals_s, cols_s, rows_s, rv, acc, sem):
    c = lax.axis_index("c")
    s = lax.axis_index("s")
    row0 = pl.multiple_of(s * RA, 8)

    @pl.when(s < NS - 1)
    def _():
        pltpu.sync_copy(z_hbm.at[pl.ds(0, RA)], acc.at[pl.ds(row0, RA)])

    @pl.when(s == NS - 1)
    def _():
        pltpu.sync_copy(z_hbm, acc.at[pl.ds((NS - 1) * RA, RLAST)])

    plsc.subcore_barrier()

    def super_body(u, carry):
        pltpu.sync_copy(mask5.at[s, u], mask_s)
        pltpu.sync_copy(vals5.at[c, s, u], vals_s)
        pltpu.sync_copy(cols5.at[c, s, u], cols_s)
        pltpu.sync_copy(rows5.at[c, s, u], rows_s)

        def scale_body(r, inner):
            for j in range(C // L):
                m = mask_s[r, pl.ds(j * L, L)]
                v = vals_s[r, pl.ds(j * L, L)]
                vals_s[r, pl.ds(j * L, L)] = v / (1.0 + jnp.exp(-m))
            return inner

        lax.fori_loop(0, CPS, scale_body, None)

        def chunk_body(k, inner):
            pltpu.async_copy(x_hbm.at[cols_s.at[k]], rv, sem).wait()

            def grp_body(g, inner2):
                sc16 = vals_s[k, pl.ds(g * L, L)]
                for i in range(L):
                    sv = _lane_broadcast(sc16, i)
                    e = g * L + i
                    for j in range(D // L):
                        rv[e, pl.ds(j * L, L)] = rv[e, pl.ds(j * L, L)] * sv
                return inner2

            lax.fori_loop(0, C // L, grp_body, None)
            pltpu.sync_copy(rv, acc.at[rows_s.at[k]], add=True)
            return inner

        lax.fori_loop(0, CPS, chunk_body, None)
        return carry

    lax.fori_loop(0, NSUP, super_body, None)
    plsc.subcore_barrier()

    @pl.when(c == 0)
    def _():
        @pl.when(s < NS - 1)
        def _():
            pltpu.sync_copy(acc.at[pl.ds(row0, RA)], hor_out.at[pl.ds(row0, RA)])

        @pl.when(s == NS - 1)
        def _():
            pltpu.sync_copy(acc.at[pl.ds((NS - 1) * RA, RLAST)],
                            hor_out.at[pl.ds((NS - 1) * RA, RLAST)])

    @pl.when(c == 1)
    def _():
        @pl.when(s < NS - 1)
        def _():
            pltpu.sync_copy(acc.at[pl.ds(row0, RA)], ver_out.at[pl.ds(row0, RA)])

        @pl.when(s == NS - 1)
        def _():
            pltpu.sync_copy(acc.at[pl.ds((NS - 1) * RA, RLAST)],
                            ver_out.at[pl.ds((NS - 1) * RA, RLAST)])


def kernel(mask, hor_values, ver_values, x, hor_indices, ver_indices):
    mask5 = mask.reshape(NS, NSUP, CPS, C)
    vals5 = jnp.stack([hor_values, ver_values]).reshape(2, NS, NSUP, CPS, C)
    cols5 = jnp.stack([hor_indices[1], ver_indices[1]]).reshape(2, NS, NSUP, CPS, C)
    rows5 = jnp.stack([hor_indices[0], ver_indices[0]]).reshape(2, NS, NSUP, CPS, C)
    z = jnp.zeros((RLAST, D), jnp.float32)
    hor_out, ver_out = _sc_propagate(mask5, vals5, cols5, rows5, x, z)
    return hor_out, ver_out


# speedup vs baseline: 10.4880x; 1.6615x over previous
import functools

import jax
import jax.numpy as jnp
from jax import lax
from jax.experimental import pallas as pl
from jax.experimental.pallas import tpu as pltpu
from jax.experimental.pallas import tpu_sc as plsc

N = 10000
E = 320000
D = 128

NS = 16
L = 16

C = 80
EPT = E // NS
CPS = 50
NSUP = EPT // (CPS * C)
RA = 624
RLAST = N - (NS - 1) * RA

_mesh = plsc.VectorSubcoreMesh(core_axis_name="c", subcore_axis_name="s")

_GATHER_DNUMS = lax.GatherDimensionNumbers(
    offset_dims=(), collapsed_slice_dims=(0,), start_index_map=(0,))


def _lane_broadcast(vec, i):
    idx = jnp.full((L, 1), i, jnp.int32)
    return lax.gather(vec, idx, _GATHER_DNUMS, slice_sizes=(1,),
                      mode=lax.GatherScatterMode.PROMISE_IN_BOUNDS)


@functools.partial(
    pl.kernel,
    out_type=[
        jax.ShapeDtypeStruct((N, D), jnp.float32),
        jax.ShapeDtypeStruct((N, D), jnp.float32),
    ],
    mesh=_mesh,
    scratch_types=[
        pltpu.VMEM((CPS, C), jnp.float32),
        pltpu.VMEM((CPS, C), jnp.float32),
        pltpu.VMEM((CPS, C), jnp.int32),
        pltpu.VMEM((CPS, C), jnp.int32),
        pltpu.VMEM((C, D), jnp.float32),
        pltpu.VMEM((C, D), jnp.float32),
        pltpu.VMEM_SHARED((N, D), jnp.float32),
        pltpu.SemaphoreType.DMA,
        pltpu.SemaphoreType.DMA,
    ],
)
def _sc_propagate(mask5, vals5, cols5, rows5, x_hbm, z_hbm, hor_out, ver_out,
                  mask_s, vals_s, cols_s, rows_s, rv0, rv1, acc, sem0, sem1):
    c = lax.axis_index("c")
    s = lax.axis_index("s")
    row0 = pl.multiple_of(s * RA, 8)

    @pl.when(s < NS - 1)
    def _():
        pltpu.sync_copy(z_hbm.at[pl.ds(0, RA)], acc.at[pl.ds(row0, RA)])

    @pl.when(s == NS - 1)
    def _():
        pltpu.sync_copy(z_hbm, acc.at[pl.ds((NS - 1) * RA, RLAST)])

    plsc.subcore_barrier()

    def super_body(u, carry):
        pltpu.sync_copy(mask5.at[s, u], mask_s)
        pltpu.sync_copy(vals5.at[c, s, u], vals_s)
        pltpu.sync_copy(cols5.at[c, s, u], cols_s)
        pltpu.sync_copy(rows5.at[c, s, u], rows_s)

        def scale_body(r, inner):
            for j in range(C // L):
                m = mask_s[r, pl.ds(j * L, L)]
                v = vals_s[r, pl.ds(j * L, L)]
                vals_s[r, pl.ds(j * L, L)] = v / (1.0 + jnp.exp(-m))
            return inner

        lax.fori_loop(0, CPS, scale_body, None)

        def slot(k, rv, sem):
            nxt, nrv, nsem = k + 1, (rv1 if rv is rv0 else rv0), \
                (sem1 if sem is sem0 else sem0)

            @pl.when(nxt < CPS)
            def _():
                pltpu.async_copy(x_hbm.at[cols_s.at[nxt]], nrv, nsem)

            pltpu.make_async_copy(x_hbm.at[pl.ds(0, C)], rv, sem).wait()

            def grp_body(g, inner2):
                sc16 = vals_s[k, pl.ds(g * L, L)]
                for i in range(L):
                    sv = _lane_broadcast(sc16, i)
                    e = g * L + i
                    for j in range(D // L):
                        rv[e, pl.ds(j * L, L)] = rv[e, pl.ds(j * L, L)] * sv
                return inner2

            lax.fori_loop(0, C // L, grp_body, None)
            pltpu.sync_copy(rv, acc.at[rows_s.at[k]], add=True)

        pltpu.async_copy(x_hbm.at[cols_s.at[0]], rv0, sem0)

        def pair_body(p, inner):
            slot(2 * p, rv0, sem0)
            slot(2 * p + 1, rv1, sem1)
            return inner

        lax.fori_loop(0, CPS // 2, pair_body, None)
        return carry

    lax.fori_loop(0, NSUP, super_body, None)
    plsc.subcore_barrier()

    @pl.when(c == 0)
    def _():
        @pl.when(s < NS - 1)
        def _():
            pltpu.sync_copy(acc.at[pl.ds(row0, RA)], hor_out.at[pl.ds(row0, RA)])

        @pl.when(s == NS - 1)
        def _():
            pltpu.sync_copy(acc.at[pl.ds((NS - 1) * RA, RLAST)],
                            hor_out.at[pl.ds((NS - 1) * RA, RLAST)])

    @pl.when(c == 1)
    def _():
        @pl.when(s < NS - 1)
        def _():
            pltpu.sync_copy(acc.at[pl.ds(row0, RA)], ver_out.at[pl.ds(row0, RA)])

        @pl.when(s == NS - 1)
        def _():
            pltpu.sync_copy(acc.at[pl.ds((NS - 1) * RA, RLAST)],
                            ver_out.at[pl.ds((NS - 1) * RA, RLAST)])


def kernel(mask, hor_values, ver_values, x, hor_indices, ver_indices):
    mask5 = mask.reshape(NS, NSUP, CPS, C)
    vals5 = jnp.stack([hor_values, ver_values]).reshape(2, NS, NSUP, CPS, C)
    cols5 = jnp.stack([hor_indices[1], ver_indices[1]]).reshape(2, NS, NSUP, CPS, C)
    rows5 = jnp.stack([hor_indices[0], ver_indices[0]]).reshape(2, NS, NSUP, CPS, C)
    z = jnp.zeros((RLAST, D), jnp.float32)
    hor_out, ver_out = _sc_propagate(mask5, vals5, cols5, rows5, x, z)
    return hor_out, ver_out


# speedup vs baseline: 11.7783x; 1.1230x over previous
import functools

import jax
import jax.numpy as jnp
from jax import lax
from jax.experimental import pallas as pl
from jax.experimental.pallas import tpu as pltpu
from jax.experimental.pallas import tpu_sc as plsc

N = 10000
E = 320000
D = 128

NS = 16
L = 16

C = 80
EPT = E // NS
CPS = 50
NSUP = EPT // (CPS * C)
RA = 624
RLAST = N - (NS - 1) * RA

_mesh = plsc.VectorSubcoreMesh(core_axis_name="c", subcore_axis_name="s")

_GATHER_DNUMS = lax.GatherDimensionNumbers(
    offset_dims=(), collapsed_slice_dims=(0,), start_index_map=(0,))


def _lane_broadcast(vec, i):
    idx = jnp.full((L, 1), i, jnp.int32)
    return lax.gather(vec, idx, _GATHER_DNUMS, slice_sizes=(1,),
                      mode=lax.GatherScatterMode.PROMISE_IN_BOUNDS)


@functools.partial(
    pl.kernel,
    out_type=[
        jax.ShapeDtypeStruct((N, D), jnp.float32),
        jax.ShapeDtypeStruct((N, D), jnp.float32),
    ],
    mesh=_mesh,
    scratch_types=[
        pltpu.VMEM((CPS * C,), jnp.float32),
        pltpu.VMEM((CPS * C,), jnp.float32),
        pltpu.VMEM((CPS * C,), jnp.int32),
        pltpu.VMEM((CPS, C), jnp.int32),
        pltpu.VMEM((C, D), jnp.float32),
        pltpu.VMEM((C, D), jnp.float32),
        pltpu.VMEM((C, D), jnp.float32),
        pltpu.VMEM_SHARED((N, D), jnp.float32),
        pltpu.SemaphoreType.DMA,
        pltpu.SemaphoreType.DMA,
        pltpu.SemaphoreType.DMA,
        pltpu.SemaphoreType.DMA,
        pltpu.SemaphoreType.DMA,
        pltpu.SemaphoreType.DMA,
    ],
)
def _sc_propagate(mask5, vals5, cols5, rows5, x_hbm, z_hbm, hor_out, ver_out,
                  mask_s, vals_s, cols_s, rows_s, rv0, rv1, rv2, acc,
                  sg0, sg1, sg2, ss0, ss1, ss2):
    c = lax.axis_index("c")
    s = lax.axis_index("s")
    row0 = pl.multiple_of(s * RA, 8)

    @pl.when(s < NS - 1)
    def _():
        pltpu.sync_copy(z_hbm.at[pl.ds(0, RA)], acc.at[pl.ds(row0, RA)])

    @pl.when(s == NS - 1)
    def _():
        pltpu.sync_copy(z_hbm, acc.at[pl.ds((NS - 1) * RA, RLAST)])

    plsc.subcore_barrier()

    def super_body(u, carry):
        pltpu.sync_copy(mask5.at[s, u], mask_s)
        pltpu.sync_copy(vals5.at[c, s, u], vals_s)
        pltpu.sync_copy(cols5.at[c, s, u], cols_s)
        pltpu.sync_copy(rows5.at[c, s, u], rows_s)

        def scale_body(r, inner):
            m = mask_s[pl.ds(r * L, L)]
            v = vals_s[pl.ds(r * L, L)]
            vals_s[pl.ds(r * L, L)] = v / (1.0 + jnp.exp(-m))
            return inner

        lax.fori_loop(0, CPS * C // L, scale_body, None)

        def scale_rows(k, rv):
            def grp_body(g, inner2):
                sc16 = vals_s[pl.ds(k * C + g * L, L)]
                for i in range(L):
                    sv = _lane_broadcast(sc16, i)
                    e = g * L + i
                    for j in range(D // L):
                        rv[e, pl.ds(j * L, L)] = rv[e, pl.ds(j * L, L)] * sv
                return inner2

            lax.fori_loop(0, C // L, grp_body, None)

        def slot(k, rv, sg, ss, nrv, nsg, nss, wait_prev_scatter):
            if wait_prev_scatter:
                pltpu.make_async_copy(nrv, acc.at[pl.ds(0, C)], nss).wait()

            @pl.when(k + 1 < CPS)
            def _():
                pltpu.async_copy(
                    x_hbm.at[cols_s.at[pl.ds((k + 1) * C, C)]], nrv, nsg)

            pltpu.make_async_copy(x_hbm.at[pl.ds(0, C)], rv, sg).wait()
            scale_rows(k, rv)
            pltpu.async_copy(rv, acc.at[rows_s.at[k]], ss, add=True)

        pltpu.async_copy(x_hbm.at[cols_s.at[pl.ds(0, C)]], rv0, sg0)
        slot(0, rv0, sg0, ss0, rv1, sg1, ss1, False)
        slot(1, rv1, sg1, ss1, rv2, sg2, ss2, False)

        def triple_body(q, inner):
            k = 3 * q + 2
            slot(k, rv2, sg2, ss2, rv0, sg0, ss0, True)
            slot(k + 1, rv0, sg0, ss0, rv1, sg1, ss1, True)
            slot(k + 2, rv1, sg1, ss1, rv2, sg2, ss2, True)
            return inner

        lax.fori_loop(0, (CPS - 2) // 3, triple_body, None)
        pltpu.make_async_copy(rv0, acc.at[pl.ds(0, C)], ss0).wait()
        pltpu.make_async_copy(rv1, acc.at[pl.ds(0, C)], ss1).wait()
        return carry

    lax.fori_loop(0, NSUP, super_body, None)
    plsc.subcore_barrier()

    @pl.when(c == 0)
    def _():
        @pl.when(s < NS - 1)
        def _():
            pltpu.sync_copy(acc.at[pl.ds(row0, RA)], hor_out.at[pl.ds(row0, RA)])

        @pl.when(s == NS - 1)
        def _():
            pltpu.sync_copy(acc.at[pl.ds((NS - 1) * RA, RLAST)],
                            hor_out.at[pl.ds((NS - 1) * RA, RLAST)])

    @pl.when(c == 1)
    def _():
        @pl.when(s < NS - 1)
        def _():
            pltpu.sync_copy(acc.at[pl.ds(row0, RA)], ver_out.at[pl.ds(row0, RA)])

        @pl.when(s == NS - 1)
        def _():
            pltpu.sync_copy(acc.at[pl.ds((NS - 1) * RA, RLAST)],
                            ver_out.at[pl.ds((NS - 1) * RA, RLAST)])


def kernel(mask, hor_values, ver_values, x, hor_indices, ver_indices):
    mask5 = mask.reshape(NS, NSUP, CPS * C)
    vals5 = jnp.stack([hor_values, ver_values]).reshape(2, NS, NSUP, CPS * C)
    cols5 = jnp.stack([hor_indices[1], ver_indices[1]]).reshape(2, NS, NSUP, CPS * C)
    rows5 = jnp.stack([hor_indices[0], ver_indices[0]]).reshape(2, NS, NSUP, CPS, C)
    z = jnp.zeros((RLAST, D), jnp.float32)
    hor_out, ver_out = _sc_propagate(mask5, vals5, cols5, rows5, x, z)
    return hor_out, ver_out
